# baseline (speedup 1.0000x reference)
import functools

import jax
import jax.numpy as jnp
from jax import lax
from jax.experimental import pallas as pl
from jax.experimental.pallas import tpu as pltpu
from jax.experimental.pallas import tpu_sc as plsc

B = 16384
D = 64
NC = 2
NS = 16
NW = NC * NS
BPW = B // NW
L = 16
CHUNKS = D // L

_mesh = plsc.VectorSubcoreMesh(core_axis_name="c", subcore_axis_name="s")


@functools.partial(
    pl.kernel,
    mesh=_mesh,
    out_type=jax.ShapeDtypeStruct((NW * L,), jnp.float32),
    scratch_types=[
        pltpu.VMEM((BPW,), jnp.int32),
        pltpu.VMEM((BPW, D), jnp.float32),
        pltpu.VMEM((D, BPW), jnp.float32),
        pltpu.VMEM((L,), jnp.float32),
        pltpu.SemaphoreType.DMA,
        pltpu.SemaphoreType.DMA,
    ],
    compiler_params=pltpu.CompilerParams(needs_layout_passes=False),
)
def _proto_loss_partials(featT_hbm, lab_hbm, proto_hbm, out_hbm,
                         idx_v, rows_v, featT_v, acc_v, gsem, fsem):
    wid = lax.axis_index("s") * NC + lax.axis_index("c")
    base = wid * BPW

    pltpu.sync_copy(lab_hbm.at[pl.ds(base, BPW)], idx_v)
    fcopy = pltpu.async_copy(featT_hbm.at[:, pl.ds(base, BPW)], featT_v,
                             fsem)

    def fire(j, _):
        idxs = idx_v[pl.ds(j * L, L)]
        for k in range(L):
            pltpu.async_copy(proto_hbm.at[idxs[k], :],
                             rows_v.at[j * L + k], gsem)
        return 0

    lax.fori_loop(0, BPW // L, fire, 0)

    def drain(i, _):
        pltpu.make_async_copy(proto_hbm.at[0, :], rows_v.at[0], gsem).wait()
        return 0

    lax.fori_loop(0, BPW, drain, 0)
    fcopy.wait()

    lane = lax.iota(jnp.int32, L)

    def body(i, accs):
        new = list(accs)
        for c in range(CHUNKS):
            p = rows_v[i, pl.ds(c * L, L)]
            ft = plsc.load_gather(featT_v, [c * L + lane,
                                            jnp.full((L,), i, jnp.int32)])
            df = ft - p
            new[c] = new[c] + df * df
        return tuple(new)

    zero = jnp.zeros((L,), jnp.float32)
    accs = lax.fori_loop(0, BPW, body, (zero,) * CHUNKS)
    acc_v[...] = (accs[0] + accs[1]) + (accs[2] + accs[3])
    pltpu.sync_copy(acc_v, out_hbm.at[pl.ds(wid * L, L)])


def kernel(features, labels, prototypes):
    partials = _proto_loss_partials(features.T, labels.astype(jnp.int32),
                                    prototypes)
    return jnp.sum(partials) * (1.0 / (B * D))
